# Initial kernel scaffold; baseline (speedup 1.0000x reference)
import functools

import jax
import jax.numpy as jnp
from jax import lax
from jax.experimental import pallas as pl
from jax.experimental.pallas import tpu as pltpu
from jax.experimental.pallas import tpu_sc as plsc

_NC = 2
_NS = 16
_NW = _NC * _NS

_NBUF = 4
_CHUNK = 256


def _body(x_hbm, table_hbm, out_hbm, idx_v, rows_v, *sems):
  n = x_hbm.shape[0]
  per_w = n // _NW
  nch = per_w // _CHUNK
  outer = nch // _NBUF
  gsems = sems[:_NBUF]
  ssems = sems[_NBUF:]

  wid = lax.axis_index("s") * _NC + lax.axis_index("c")
  base = pl.multiple_of(wid * per_w, per_w)

  pltpu.sync_copy(x_hbm.at[pl.ds(base, per_w)], idx_v)

  def gather(i, b):
    return pltpu.async_copy(
        table_hbm.at[idx_v.at[pl.ds(i * _CHUNK, _CHUNK)]], rows_v.at[b],
        gsems[b])

  def store(i, b):
    return pltpu.async_copy(
        rows_v.at[b], out_hbm.at[pl.ds(base + i * _CHUNK, _CHUNK)], ssems[b])

  for b in range(_NBUF):
    gather(b, b)

  @pl.loop(0, outer - 1)
  def _(o):
    i0 = pl.multiple_of(o * _NBUF, _NBUF)
    for b in range(_NBUF):
      i = i0 + b
      gather(i, b).wait()
      store(i, b)
      store(i, b).wait()
      gather(i + _NBUF, b)

  i0 = (outer - 1) * _NBUF
  for b in range(_NBUF):
    i = i0 + b
    gather(i, b).wait()
    store(i, b)
  for b in range(_NBUF):
    store(i0 + b, b).wait()


def _run(x_flat, table):
  n = x_flat.shape[0]
  d = table.shape[1]
  per_w = n // _NW
  mesh = plsc.VectorSubcoreMesh(core_axis_name="c", subcore_axis_name="s")
  sems = [pltpu.SemaphoreType.DMA] * (2 * _NBUF)
  return pl.kernel(
      _body,
      out_type=jax.ShapeDtypeStruct((n, d), table.dtype),
      mesh=mesh,
      scratch_types=[
          pltpu.VMEM((per_w,), jnp.int32),
          pltpu.VMEM((_NBUF, _CHUNK, d), table.dtype),
      ] + sems,
  )(x_flat, table)


@jax.jit
def kernel(x, table):
  b, h = x.shape
  out = _run(x.reshape(b * h).astype(jnp.int32), table)
  return out.reshape(b, h, table.shape[1])


# speedup vs baseline: 1.8739x; 1.8739x over previous
import functools

import jax
import jax.numpy as jnp
from jax import lax
from jax.experimental import pallas as pl
from jax.experimental.pallas import tpu as pltpu
from jax.experimental.pallas import tpu_sc as plsc

_NC = 2
_NS = 16
_NW = _NC * _NS

_NBUF = 4
_CHUNK = 256


def _body(x_hbm, table_hbm, out_hbm, idx_v, rows_v, *sems):
  n = x_hbm.shape[0]
  per_w = n // _NW
  nch = per_w // _CHUNK
  outer = nch // _NBUF
  gsems = sems[:_NBUF]
  ssems = sems[_NBUF:]

  wid = lax.axis_index("s") * _NC + lax.axis_index("c")
  base = pl.multiple_of(wid * per_w, per_w)

  pltpu.sync_copy(x_hbm.at[pl.ds(base, per_w)], idx_v)

  def gather(i, b):
    return pltpu.make_async_copy(
        table_hbm.at[idx_v.at[pl.ds(i * _CHUNK, _CHUNK)]], rows_v.at[b],
        gsems[b])

  def store(i, b):
    return pltpu.make_async_copy(
        rows_v.at[b], out_hbm.at[pl.ds(base + i * _CHUNK, _CHUNK)], ssems[b])

  for b in range(_NBUF):
    gather(b, b).start()

  @pl.loop(0, outer - 1)
  def _(o):
    i0 = pl.multiple_of(o * _NBUF, _NBUF)
    for b in range(_NBUF):
      i = i0 + b
      gather(i, b).wait()
      st = store(i, b)
      st.start()
      st.wait()
      gather(i + _NBUF, b).start()

  i0 = (outer - 1) * _NBUF
  stores = []
  for b in range(_NBUF):
    i = i0 + b
    gather(i, b).wait()
    st = store(i, b)
    st.start()
    stores.append(st)
  for st in stores:
    st.wait()


def _run(x_flat, table):
  n = x_flat.shape[0]
  d = table.shape[1]
  per_w = n // _NW
  mesh = plsc.VectorSubcoreMesh(core_axis_name="c", subcore_axis_name="s")
  sems = [pltpu.SemaphoreType.DMA] * (2 * _NBUF)
  return pl.kernel(
      _body,
      out_type=jax.ShapeDtypeStruct((n, d), table.dtype),
      mesh=mesh,
      compiler_params=pltpu.CompilerParams(use_tc_tiling_on_sc=False),
      scratch_types=[
          pltpu.VMEM((per_w,), jnp.int32),
          pltpu.VMEM((_NBUF, _CHUNK, d), table.dtype),
      ] + sems,
  )(x_flat, table)


@jax.jit
def kernel(x, table):
  b, h = x.shape
  out = _run(x.reshape(b * h).astype(jnp.int32), table)
  return out.reshape(b, h, table.shape[1])


# speedup vs baseline: 1.8756x; 1.0009x over previous
import functools

import jax
import jax.numpy as jnp
from jax import lax
from jax.experimental import pallas as pl
from jax.experimental.pallas import tpu as pltpu
from jax.experimental.pallas import tpu_sc as plsc

_NC = 2
_NS = 16
_NW = _NC * _NS

_NBUF = 8
_CHUNK = 128


def _body(x_hbm, table_hbm, out_hbm, idx_v, rows_v, *sems):
  n = x_hbm.shape[0]
  per_w = n // _NW
  nch = per_w // _CHUNK
  outer = nch // _NBUF
  gsems = sems[:_NBUF]
  ssems = sems[_NBUF:]

  wid = lax.axis_index("s") * _NC + lax.axis_index("c")
  base = pl.multiple_of(wid * per_w, per_w)

  pltpu.sync_copy(x_hbm.at[pl.ds(base, per_w)], idx_v)

  def gather(i, b):
    return pltpu.make_async_copy(
        table_hbm.at[idx_v.at[pl.ds(i * _CHUNK, _CHUNK)]], rows_v.at[b],
        gsems[b])

  def store(i, b):
    return pltpu.make_async_copy(
        rows_v.at[b], out_hbm.at[pl.ds(base + i * _CHUNK, _CHUNK)], ssems[b])

  for b in range(_NBUF):
    gather(b, b).start()

  @pl.loop(0, outer - 1)
  def _(o):
    i0 = pl.multiple_of(o * _NBUF, _NBUF)
    for b in range(_NBUF):
      i = i0 + b
      gather(i, b).wait()
      st = store(i, b)
      st.start()
      st.wait()
      gather(i + _NBUF, b).start()

  i0 = (outer - 1) * _NBUF
  stores = []
  for b in range(_NBUF):
    i = i0 + b
    gather(i, b).wait()
    st = store(i, b)
    st.start()
    stores.append(st)
  for st in stores:
    st.wait()


def _run(x_flat, table):
  n = x_flat.shape[0]
  d = table.shape[1]
  per_w = n // _NW
  mesh = plsc.VectorSubcoreMesh(core_axis_name="c", subcore_axis_name="s")
  sems = [pltpu.SemaphoreType.DMA] * (2 * _NBUF)
  return pl.kernel(
      _body,
      out_type=jax.ShapeDtypeStruct((n, d), table.dtype),
      mesh=mesh,
      compiler_params=pltpu.CompilerParams(use_tc_tiling_on_sc=False),
      scratch_types=[
          pltpu.VMEM((per_w,), jnp.int32),
          pltpu.VMEM((_NBUF, _CHUNK, d), table.dtype),
      ] + sems,
  )(x_flat, table)


@jax.jit
def kernel(x, table):
  b, h = x.shape
  out = _run(x.reshape(b * h).astype(jnp.int32), table)
  return out.reshape(b, h, table.shape[1])
